# baseline (speedup 1.0000x reference)
import functools

import jax
import jax.numpy as jnp
import numpy as np
from jax import lax
from jax.experimental import pallas as pl
from jax.experimental.pallas import tpu as pltpu
from jax.experimental.pallas import tpu_sc as plsc

M_MAX = 2
MAX_NO_ORBITALS_PER_M = 2
MAX_SPLIT_PER_M = 1
NCOEF = 4
NO_ORB = M_MAX * MAX_NO_ORBITALS_PER_M * MAX_SPLIT_PER_M
P = NO_ORB * (NO_ORB - 1) // 2
G = 16

BLK = 1280
EBLK = 1280

_I_IDX, _J_IDX = np.triu_indices(NO_ORB, k=1)


def _emb_body(nblk_valid, x_ref, w_ref, b_ref, o_ref):
    i = pl.program_id(0)

    @pl.when(i < nblk_valid)
    def _():
        y = (
            jnp.dot(x_ref[...], w_ref[...], preferred_element_type=jnp.float32)
            + b_ref[...]
        )
        o_ref[0, 0] = jnp.transpose(y)

    @pl.when(i >= nblk_valid)
    def _():
        o_ref[...] = jnp.zeros_like(o_ref)


def _make_emb(n_pairs, n_pad, emb_size):
    assert n_pairs % EBLK == 0 and (n_pad // 2) % EBLK == 0
    nblk_valid = n_pairs // EBLK
    grid = n_pad // EBLK
    per_half = grid // 2
    half = n_pad // 2
    return pl.pallas_call(
        functools.partial(_emb_body, nblk_valid),
        grid=(grid,),
        in_specs=[
            pl.BlockSpec((EBLK, emb_size), lambda i: (jnp.minimum(i, nblk_valid - 1), 0)),
            pl.BlockSpec((emb_size, P), lambda i: (0, 0)),
            pl.BlockSpec((1, P), lambda i: (0, 0)),
        ],
        out_specs=pl.BlockSpec(
            (1, 1, P, EBLK), lambda i: (i // per_half, i % per_half, 0, 0)
        ),
        out_shape=jax.ShapeDtypeStruct((2, per_half, P, EBLK), jnp.float32),
    )


def _table_body(z_ref, r_ref, c_ref, o_ref):
    zf = z_ref[...]
    r = r_ref[...]
    coords = c_ref[...]
    d = r[:, None, :] - coords[None, :, :]
    dist = jnp.sqrt(jnp.sum(d * d, axis=-1))
    mult = 1.0 + 0.01 * jnp.sqrt(jnp.sum(r * r, axis=-1))
    o_i = lax.broadcasted_iota(jnp.int32, (NO_ORB, NCOEF), 0)
    c_i = lax.broadcasted_iota(jnp.int32, (NO_ORB, NCOEF), 1)
    arr = 0.5 + 0.1 * (NCOEF * o_i + c_i).astype(jnp.float32)
    base = 0.1 * (zf[:, 0] + 1.0) * mult
    coeff = base[:, None, None] * arr[None, :, :]
    earg = coeff[:, :, None, :] * dist[:, None, :, None]
    orb = jnp.sum(coeff[:, :, None, :] * jnp.exp(-earg), axis=-1)
    chans = [orb[:, int(i), :] * orb[:, int(j), :] for i, j in zip(_I_IDX, _J_IDX)]
    op = jnp.concatenate(chans, axis=-1)
    op = op.reshape(op.shape[0], P, G)
    op = jnp.transpose(op, (0, 2, 1)).reshape(op.shape[0], G * P)
    o_ref[...] = jnp.transpose(op)


def _make_table(n_atoms):
    blk = 1280
    assert n_atoms % blk == 0
    grid = n_atoms // blk
    return pl.pallas_call(
        _table_body,
        grid=(grid,),
        in_specs=[
            pl.BlockSpec((blk, 1), lambda i: (i, 0)),
            pl.BlockSpec((blk, 3), lambda i: (i, 0)),
            pl.BlockSpec((G, 3), lambda i: (0, 0)),
        ],
        out_specs=pl.BlockSpec((G * P, blk), lambda i: (0, i)),
        out_shape=jax.ShapeDtypeStruct((G * P, n_atoms), jnp.float32),
    )


def _sc_body(n_blocks, n_mol, cols_hbm, idxi_hbm, idxj_hbm, mol_hbm, emb_hbm,
             out_hbm, cols_v, idxi_v, idxj_v, mol_v, emb_v, acc_v, sem):
    cid = lax.axis_index("c")
    sid = lax.axis_index("s")

    pltpu.sync_copy(cols_hbm.at[sid], cols_v)

    zero = jnp.zeros((16,), jnp.float32)
    zeros6 = (zero,) * P

    def zinit(m, carry):
        for p in range(P):
            acc_v[pl.ds(m * (P * 16) + 16 * p, 16)] = zero
        return carry

    lax.fori_loop(0, n_mol, zinit, 0)

    lane = lax.iota(jnp.int32, 16)

    def group(k, carry):
        accs = carry[:P]
        prev = carry[P]
        base = k * 16
        molv = mol_v[pl.ds(base, 16)]
        i16 = idxi_v[pl.ds(base, 16)]
        j16 = idxj_v[pl.ds(base, 16)]
        prods = []
        for p in range(P):
            vi = plsc.load_gather(cols_v.at[p], [i16])
            vj = plsc.load_gather(cols_v.at[p], [j16])
            ep = emb_v[p, pl.ds(base, 16)]
            prods.append(vi * vj * ep)

        same = jnp.all(molv == prev)

        def stay(_):
            return tuple(a + q for a, q in zip(accs, prods)) + (prev,)

        def boundary(_):
            m_sc = jnp.maximum(prev[0], 0)
            for p in range(P):
                acc_v[pl.ds(m_sc * (P * 16) + 16 * p, 16)] += accs[p]
            uni = jnp.all(molv == jnp.flip(molv, 0))
            nprev = jnp.full((16,), jnp.max(molv), jnp.int32)

            def fastgrp(_):
                return tuple(prods) + (nprev,)

            def slowgrp(_):
                for p in range(P):
                    idxs = molv * (P * 16) + 16 * p + lane
                    plsc.addupdate_scatter(acc_v, [idxs], prods[p])
                return zeros6 + (nprev,)

            return lax.cond(uni, fastgrp, slowgrp, 0)

        return lax.cond(same, stay, boundary, 0)

    def block(bi, carry):
        b0 = bi * BLK
        pltpu.sync_copy(idxi_hbm.at[cid, pl.ds(b0, BLK)], idxi_v)
        pltpu.sync_copy(idxj_hbm.at[cid, pl.ds(b0, BLK)], idxj_v)
        pltpu.sync_copy(mol_hbm.at[cid, pl.ds(b0, BLK)], mol_v)
        pltpu.sync_copy(emb_hbm.at[cid, bi], emb_v)
        return lax.fori_loop(0, BLK // 16, group, carry)

    prev0 = jnp.full((16,), -1, jnp.int32)
    carry = lax.fori_loop(0, n_blocks, block, zeros6 + (prev0,))

    m_sc = jnp.maximum(carry[P][0], 0)
    for p in range(P):
        acc_v[pl.ds(m_sc * (P * 16) + 16 * p, 16)] += carry[p]

    pltpu.sync_copy(acc_v, out_hbm.at[cid, sid])


def _make_sc(n_atoms, n_blocks, n_mol):
    mesh = plsc.VectorSubcoreMesh(
        core_axis_name="c", subcore_axis_name="s", num_cores=2, num_subcores=16
    )
    return pl.kernel(
        functools.partial(_sc_body, n_blocks, n_mol),
        out_type=jax.ShapeDtypeStruct((2, G, n_mol * P * 16), jnp.float32),
        mesh=mesh,
        scratch_types=[
            pltpu.VMEM((P, n_atoms), jnp.float32),
            pltpu.VMEM((BLK,), jnp.int32),
            pltpu.VMEM((BLK,), jnp.int32),
            pltpu.VMEM((BLK,), jnp.int32),
            pltpu.VMEM((P, BLK), jnp.float32),
            pltpu.VMEM((n_mol * P * 16,), jnp.float32),
            pltpu.SemaphoreType.DMA,
        ],
        compiler_params=pltpu.CompilerParams(
            use_tc_tiling_on_sc=False, needs_layout_passes=False
        ),
    )


def _reduce_body(p_ref, o_ref):
    o_ref[...] = jnp.sum(p_ref[...], axis=(0, 2))[None, :]


def _make_reduce(n_mol):
    return pl.pallas_call(
        _reduce_body,
        out_shape=jax.ShapeDtypeStruct((1, G * n_mol * P), jnp.float32),
    )


def kernel(out, Z, R, coords, N, atom_pair_indices, atom_pair_mol_id, W, b):
    n_pairs, emb_size = out.shape
    n_atoms = Z.shape[0]
    n_mol = N.shape[0]

    round_to = 2 * BLK
    n_pad = ((n_pairs + round_to - 1) // round_to) * round_to
    half = n_pad // 2
    n_blocks = half // BLK

    embT = _make_emb(n_pairs, n_pad, emb_size)(out, W, b.reshape(1, P))
    a_pad = ((n_atoms + 1279) // 1280) * 1280
    zp = jnp.pad(Z.astype(jnp.float32), (0, a_pad - n_atoms)).reshape(a_pad, 1)
    rp = jnp.pad(R, ((0, a_pad - n_atoms), (0, 0)))
    colsT = _make_table(a_pad)(zp, rp, coords)
    cols = colsT.reshape(G, P, a_pad)

    pad_n = n_pad - n_pairs
    idx = jnp.pad(atom_pair_indices, ((0, pad_n), (0, 0)))
    idxi = idx[:, 0].reshape(2, half)
    idxj = idx[:, 1].reshape(2, half)
    mol = jnp.pad(atom_pair_mol_id, (0, pad_n)).reshape(2, half)

    partials = _make_sc(a_pad, n_blocks, n_mol)(cols, idxi, idxj, mol, embT)
    red = _make_reduce(n_mol)(partials.reshape(2, G * n_mol * P, 16))
    dens = red.reshape(G, n_mol, P)
    return jnp.transpose(dens, (1, 0, 2))


# speedup vs baseline: 1.1623x; 1.1623x over previous
import functools

import jax
import jax.numpy as jnp
import numpy as np
from jax import lax
from jax.experimental import pallas as pl
from jax.experimental.pallas import tpu as pltpu
from jax.experimental.pallas import tpu_sc as plsc

M_MAX = 2
MAX_NO_ORBITALS_PER_M = 2
MAX_SPLIT_PER_M = 1
NCOEF = 4
NO_ORB = M_MAX * MAX_NO_ORBITALS_PER_M * MAX_SPLIT_PER_M
P = NO_ORB * (NO_ORB - 1) // 2
G = 16

NW = 32
CHUNK = 64
UNROLL = 8
PE = 8

_I_IDX, _J_IDX = np.triu_indices(NO_ORB, k=1)


def _emb_body(nblk_valid, x_ref, w_ref, b_ref, o_ref):
    i = pl.program_id(0)

    @pl.when(i < nblk_valid)
    def _():
        o_ref[...] = (
            jnp.dot(x_ref[...], w_ref[...], preferred_element_type=jnp.float32)
            + b_ref[...]
        )

    @pl.when(i >= nblk_valid)
    def _():
        o_ref[...] = jnp.zeros_like(o_ref)


def _make_emb(n_pairs, n_pad, emb_size):
    blk = 1280
    assert n_pairs % blk == 0 and n_pad % blk == 0
    nblk_valid = n_pairs // blk
    grid = n_pad // blk
    return pl.pallas_call(
        functools.partial(_emb_body, nblk_valid),
        grid=(grid,),
        in_specs=[
            pl.BlockSpec((blk, emb_size), lambda i: (jnp.minimum(i, nblk_valid - 1), 0)),
            pl.BlockSpec((emb_size, PE), lambda i: (0, 0)),
            pl.BlockSpec((1, PE), lambda i: (0, 0)),
        ],
        out_specs=pl.BlockSpec((blk, PE), lambda i: (i, 0)),
        out_shape=jax.ShapeDtypeStruct((n_pad, PE), jnp.float32),
    )


def _table_body(z_ref, r_ref, c_ref, o_ref):
    zf = z_ref[...]
    r = r_ref[...]
    coords = c_ref[...]
    d = r[:, None, :] - coords[None, :, :]
    dist = jnp.sqrt(jnp.sum(d * d, axis=-1))
    mult = 1.0 + 0.01 * jnp.sqrt(jnp.sum(r * r, axis=-1))
    o_i = lax.broadcasted_iota(jnp.int32, (NO_ORB, NCOEF), 0)
    c_i = lax.broadcasted_iota(jnp.int32, (NO_ORB, NCOEF), 1)
    arr = 0.5 + 0.1 * (NCOEF * o_i + c_i).astype(jnp.float32)
    base = 0.1 * (zf[:, 0] + 1.0) * mult
    coeff = base[:, None, None] * arr[None, :, :]
    earg = coeff[:, :, None, :] * dist[:, None, :, None]
    orb = jnp.sum(coeff[:, :, None, :] * jnp.exp(-earg), axis=-1)
    chans = [orb[:, int(i), :] * orb[:, int(j), :] for i, j in zip(_I_IDX, _J_IDX)]
    o_ref[...] = jnp.concatenate(chans, axis=-1)


def _make_table(n_atoms):
    blk = 400
    assert n_atoms % blk == 0
    grid = n_atoms // blk
    return pl.pallas_call(
        _table_body,
        grid=(grid,),
        in_specs=[
            pl.BlockSpec((blk, 1), lambda i: (i, 0)),
            pl.BlockSpec((blk, 3), lambda i: (i, 0)),
            pl.BlockSpec((G, 3), lambda i: (0, 0)),
        ],
        out_specs=pl.BlockSpec((blk, P * G), lambda i: (i, 0)),
        out_shape=jax.ShapeDtypeStruct((n_atoms, P * G), jnp.float32),
    )


def _sc_body(n_chunks, n_mol, n_atoms, table_hbm, idx_hbm, mol_hbm, emb_hbm,
             out_hbm, table_sh, idx_v, mol_v, emb_v, rows_v, acc_v, sem):
    sid = lax.axis_index("s")
    wid = sid * 2 + lax.axis_index("c")

    rows_per = n_atoms // 16
    pltpu.sync_copy(
        table_hbm.at[pl.ds(sid * rows_per, rows_per)],
        table_sh.at[pl.ds(sid * rows_per, rows_per)],
    )

    tp = n_chunks * CHUNK
    pltpu.sync_copy(idx_hbm.at[wid], idx_v)
    pltpu.sync_copy(mol_hbm.at[wid], mol_v.at[pl.ds(0, tp)])
    plsc.subcore_barrier()

    zero = jnp.zeros((16,), jnp.float32)

    def zinit(m, carry):
        for p in range(P):
            acc_v[m, pl.ds(16 * p, 16)] = zero
        return carry

    lax.fori_loop(0, n_mol, zinit, 0)

    def compute(ci, rows_v, emb_v):
        g0 = ci * CHUNK
        mol_lo = mol_v[pl.ds(g0, 16)][0]
        mol_hi = mol_v[pl.ds(g0 + CHUNK - 1, 16)][0]

        def one_pair(k, accs):
            e_vec = emb_v[pl.ds(k * PE, 16)]
            o = []
            for p in range(P):
                vi = rows_v[2 * k, pl.ds(16 * p, 16)]
                vj = rows_v[2 * k + 1, pl.ds(16 * p, 16)]
                o.append(accs[p] + vi * vj * e_vec[p])
            return tuple(o)

        def fast(_):
            def grp(t, accs):
                for u in range(UNROLL):
                    accs = one_pair(t * UNROLL + u, accs)
                return accs

            accs = lax.fori_loop(0, CHUNK // UNROLL, grp, (zero,) * P)
            for p in range(P):
                acc_v[mol_lo, pl.ds(16 * p, 16)] += accs[p]
            return 0

        def slow(_):
            def pair_body(k, c2):
                g = g0 + k
                mol = mol_v[pl.ds(g, 16)][0]
                e_vec = emb_v[pl.ds(k * PE, 16)]
                for p in range(P):
                    vi = rows_v[2 * k, pl.ds(16 * p, 16)]
                    vj = rows_v[2 * k + 1, pl.ds(16 * p, 16)]
                    acc_v[mol, pl.ds(16 * p, 16)] += vi * vj * e_vec[p]
                return c2

            lax.fori_loop(0, CHUNK, pair_body, 0)
            return 0

        lax.cond(mol_lo == mol_hi, fast, slow, 0)

    def chunk_body(ci, carry):
        pltpu.async_copy(table_sh.at[idx_v.at[ci]], rows_v, sem).wait()
        pltpu.async_copy(
            emb_hbm.at[wid].at[ci], emb_v.at[pl.ds(0, CHUNK * PE)], sem
        ).wait()
        compute(ci, rows_v, emb_v)
        return carry

    lax.fori_loop(0, n_chunks, chunk_body, 0)
    pltpu.sync_copy(acc_v, out_hbm.at[wid])


def _make_sc(n_atoms, n_chunks, n_mol):
    tp = n_chunks * CHUNK
    mesh = plsc.VectorSubcoreMesh(
        core_axis_name="c", subcore_axis_name="s", num_cores=2, num_subcores=16
    )
    assert n_atoms % 16 == 0
    return pl.kernel(
        functools.partial(_sc_body, n_chunks, n_mol, n_atoms),
        out_type=jax.ShapeDtypeStruct((NW, n_mol, P * G), jnp.float32),
        mesh=mesh,
        scratch_types=[
            pltpu.VMEM_SHARED((n_atoms, P * G), jnp.float32),
            pltpu.VMEM((n_chunks, 2 * CHUNK), jnp.int32),
            pltpu.VMEM((tp + 16,), jnp.int32),
            pltpu.VMEM((CHUNK * PE + 16,), jnp.float32),
            pltpu.VMEM((2 * CHUNK, P * G), jnp.float32),
            pltpu.VMEM((n_mol, P * G), jnp.float32),
            pltpu.SemaphoreType.DMA,
        ],
        compiler_params=pltpu.CompilerParams(use_tc_tiling_on_sc=False),
    )


def _reduce_body(p_ref, o_ref):
    o_ref[...] = jnp.sum(p_ref[...], axis=0)


def _make_reduce(n_mol):
    return pl.pallas_call(
        _reduce_body,
        out_shape=jax.ShapeDtypeStruct((n_mol, P * G), jnp.float32),
    )


def kernel(out, Z, R, coords, N, atom_pair_indices, atom_pair_mol_id, W, b):
    n_pairs, emb_size = out.shape
    n_atoms = Z.shape[0]
    n_mol = N.shape[0]

    round_to = 40960
    n_pad = ((n_pairs + round_to - 1) // round_to) * round_to
    tp = n_pad // NW
    n_chunks = tp // CHUNK

    w_pad = jnp.zeros((emb_size, PE), jnp.float32).at[:, :P].set(W)
    b_pad = jnp.zeros((1, PE), jnp.float32).at[0, :P].set(b)

    emb = _make_emb(n_pairs, n_pad, emb_size)(out, w_pad, b_pad)
    table = _make_table(n_atoms)(
        Z.astype(jnp.float32).reshape(n_atoms, 1), R, coords
    )

    pad_n = n_pad - n_pairs
    idx = jnp.pad(atom_pair_indices.reshape(-1), (0, 2 * pad_n)).reshape(
        NW, n_chunks, 2 * CHUNK
    )
    mol = jnp.pad(atom_pair_mol_id, (0, pad_n)).reshape(NW, tp)
    embr = emb.reshape(NW, tp // CHUNK, CHUNK * PE)

    partials = _make_sc(n_atoms, n_chunks, n_mol)(table, idx, mol, embr)
    dens = _make_reduce(n_mol)(partials)
    return jnp.transpose(dens.reshape(n_mol, P, G), (0, 2, 1))
